# baseline (speedup 1.0000x reference)
import functools

import jax
import jax.numpy as jnp
from jax import lax
from jax.experimental import pallas as pl
from jax.experimental.pallas import tpu as pltpu
from jax.experimental.pallas import tpu_sc as plsc

N = 10000
D = 128
E = 320000

NC = 2
NS = 16
NW = NC * NS

CHUNK = 128
CPW = 78
EPW = CPW * CHUNK
TAIL = NW * EPW
NTAIL = (E - TAIL) // CHUNK
ZR = 632
N_ACC = NS * ZR

_mesh = plsc.VectorSubcoreMesh(core_axis_name="c", subcore_axis_name="s")


N_HR = 128


@functools.partial(
    pl.kernel,
    out_type=jax.ShapeDtypeStruct((NC, N_HR, 128), jnp.float32),
    mesh=_mesh,
    scratch_types=[
        pltpu.VMEM((EPW + CHUNK,), jnp.int32),
        pltpu.VMEM((N_HR, 128), jnp.float32),
        pltpu.VMEM((N_HR,), jnp.int32),
        pltpu.VMEM_SHARED((N_HR, 128), jnp.float32),
    ],
    compiler_params=pltpu.CompilerParams(needs_layout_passes=False),
)
def _deg_kernel(dst_hbm, out_hbm, idx_v, hist_v, id_v, hist_sh):
    cid = lax.axis_index("c")
    sid = lax.axis_index("s")
    wid = sid * NC + cid
    pltpu.sync_copy(dst_hbm.at[pl.ds(wid * EPW, EPW)], idx_v.at[pl.ds(0, EPW)])

    @pl.when(wid < NTAIL)
    def _():
        pltpu.sync_copy(
            dst_hbm.at[pl.ds(TAIL + wid * CHUNK, CHUNK)],
            idx_v.at[pl.ds(EPW, CHUNK)],
        )

    zeros16 = jnp.zeros((16,), jnp.float32)
    ones16 = jnp.ones((16,), jnp.float32)

    @pl.loop(0, N_HR)
    def _(i):
        for j in range(8):
            hist_v[i, pl.ds(j * 16, 16)] = zeros16

    @pl.loop(0, N_HR // 16)
    def _(k):
        id_v[pl.ds(k * 16, 16)] = lax.iota(jnp.int32, 16) + k * 16

    @pl.when(sid == 0)
    def _():
        pltpu.sync_copy(hist_v, hist_sh)

    plsc.subcore_barrier()

    def add_hist(e):
        v = idx_v[pl.ds(e * 16, 16)]
        plsc.addupdate_scatter(
            hist_v,
            [lax.shift_right_logical(v, 7), jnp.bitwise_and(v, 127)],
            ones16,
        )

    @pl.loop(0, EPW // 16)
    def _(e):
        add_hist(e)

    @pl.when(wid < NTAIL)
    def _():
        @pl.loop(EPW // 16, (EPW + CHUNK) // 16)
        def _(e):
            add_hist(e)

    pltpu.sync_copy(hist_v, hist_sh.at[id_v], add=True)
    plsc.subcore_barrier()
    pltpu.sync_copy(
        hist_sh.at[pl.ds(sid * 8, 8)], out_hbm.at[cid, pl.ds(sid * 8, 8)]
    )


@functools.partial(
    pl.kernel,
    out_type=jax.ShapeDtypeStruct((NC, N_ACC, D), jnp.float32),
    mesh=_mesh,
    scratch_types=[
        pltpu.VMEM((EPW + CHUNK,), jnp.int32),
        pltpu.VMEM((CHUNK,), jnp.int32),
        pltpu.VMEM((CHUNK,), jnp.int32),
        pltpu.VMEM((CHUNK, D), jnp.float32),
        pltpu.VMEM((CHUNK, D), jnp.float32),
        pltpu.VMEM_SHARED((N_ACC, D), jnp.float32),
        pltpu.SemaphoreType.DMA,
        pltpu.SemaphoreType.DMA,
    ],
)
def _agg_kernel(src_hbm, dst_hbm, g_hbm, zeros_hbm, out_hbm,
                src_v, dst_a, dst_b, rows_a, rows_b, acc_sh, sem_a, sem_b):
    cid = lax.axis_index("c")
    sid = lax.axis_index("s")
    wid = sid * NC + cid
    base = wid * EPW
    pltpu.sync_copy(src_hbm.at[pl.ds(base, EPW)], src_v.at[pl.ds(0, EPW)])

    @pl.when(wid < NTAIL)
    def _():
        pltpu.sync_copy(
            src_hbm.at[pl.ds(TAIL + wid * CHUNK, CHUNK)],
            src_v.at[pl.ds(EPW, CHUNK)],
        )

    pltpu.sync_copy(zeros_hbm, acc_sh.at[pl.ds(sid * ZR, ZR)])
    plsc.subcore_barrier()

    def gather(i, rows, sem):
        return pltpu.make_async_copy(
            g_hbm.at[src_v.at[pl.ds(i * CHUNK, CHUNK)]], rows, sem
        )

    def load_dst(i, dst_buf):
        pltpu.sync_copy(dst_hbm.at[pl.ds(base + i * CHUNK, CHUNK)], dst_buf)

    load_dst(0, dst_a)
    gather(0, rows_a, sem_a).start()

    @pl.loop(0, CPW // 2)
    def _(k):
        i = k * 2
        load_dst(i + 1, dst_b)
        gather(i + 1, rows_b, sem_b).start()
        gather(i, rows_a, sem_a).wait()
        pltpu.sync_copy(rows_a, acc_sh.at[dst_a], add=True)

        @pl.when(k < CPW // 2 - 1)
        def _():
            load_dst(i + 2, dst_a)
            gather(i + 2, rows_a, sem_a).start()

        gather(i + 1, rows_b, sem_b).wait()
        pltpu.sync_copy(rows_b, acc_sh.at[dst_b], add=True)

    @pl.when(wid < NTAIL)
    def _():
        pltpu.sync_copy(dst_hbm.at[pl.ds(TAIL + wid * CHUNK, CHUNK)], dst_a)
        gather(CPW, rows_a, sem_a).start()
        gather(CPW, rows_a, sem_a).wait()
        pltpu.sync_copy(rows_a, acc_sh.at[dst_a], add=True)

    plsc.subcore_barrier()
    pltpu.sync_copy(
        acc_sh.at[pl.ds(sid * ZR, ZR)], out_hbm.at[cid, pl.ds(sid * ZR, ZR)]
    )


def _scale_body(x_ref, w_ref, d0_ref, d1_ref, g_ref, dinv_ref):
    h = jnp.dot(x_ref[...], w_ref[...], preferred_element_type=jnp.float32)
    dinv = lax.rsqrt(d0_ref[...] + d1_ref[...] + 1.0)
    dinv_ref[...] = dinv
    g_ref[...] = h * dinv


def _epilogue_body(dinv_ref, g_ref, acc_ref, b_ref, o_ref):
    o_ref[...] = (
        dinv_ref[...] * (acc_ref[0] + acc_ref[1] + g_ref[...]) + b_ref[...]
    )


_BR = 2000


def kernel(x, edge_index, W, b):
    src = edge_index[0].astype(jnp.int32)
    dst = edge_index[1].astype(jnp.int32)

    zerosD = jnp.zeros((ZR, D), jnp.float32)

    deg = _deg_kernel(dst).reshape(NC, N_HR * 128)
    d0 = deg[0, :N, None]
    d1 = deg[1, :N, None]

    grid = (N // _BR,)
    row_spec = pl.BlockSpec((_BR, D), lambda i: (i, 0))
    col_spec = pl.BlockSpec((_BR, 1), lambda i: (i, 0))
    g, dinv = pl.pallas_call(
        _scale_body,
        grid=grid,
        in_specs=[
            row_spec,
            pl.BlockSpec((D, D), lambda i: (0, 0)),
            col_spec,
            col_spec,
        ],
        out_specs=[row_spec, col_spec],
        out_shape=[
            jax.ShapeDtypeStruct((N, D), jnp.float32),
            jax.ShapeDtypeStruct((N, 1), jnp.float32),
        ],
    )(x, W, d0, d1)

    acc = _agg_kernel(src, dst, g, zerosD)

    out = pl.pallas_call(
        _epilogue_body,
        grid=grid,
        in_specs=[
            col_spec,
            row_spec,
            pl.BlockSpec((NC, _BR, D), lambda i: (0, i, 0)),
            pl.BlockSpec((1, D), lambda i: (0, 0)),
        ],
        out_specs=row_spec,
        out_shape=jax.ShapeDtypeStruct((N, D), jnp.float32),
    )(dinv, g, acc, b.reshape(1, D))
    return out


# speedup vs baseline: 1.0789x; 1.0789x over previous
import functools

import jax
import jax.numpy as jnp
from jax import lax
from jax.experimental import pallas as pl
from jax.experimental.pallas import tpu as pltpu
from jax.experimental.pallas import tpu_sc as plsc

N = 10000
D = 128
E = 320000

NC = 2
NS = 16
NW = NC * NS

CHUNK = 128
CPW = 78
EPW = CPW * CHUNK
TAIL = NW * EPW
NTAIL = (E - TAIL) // CHUNK
ZR = 632
N_ACC = NS * ZR

_mesh = plsc.VectorSubcoreMesh(core_axis_name="c", subcore_axis_name="s")


N_HR = 128


@functools.partial(
    pl.kernel,
    out_type=jax.ShapeDtypeStruct((NC, N_HR, 128), jnp.float32),
    mesh=_mesh,
    scratch_types=[
        pltpu.VMEM((EPW + CHUNK,), jnp.int32),
        pltpu.VMEM((N_HR, 128), jnp.float32),
        pltpu.VMEM((N_HR,), jnp.int32),
        pltpu.VMEM_SHARED((N_HR, 128), jnp.float32),
    ],
    compiler_params=pltpu.CompilerParams(needs_layout_passes=False),
)
def _deg_kernel(ei_hbm, out_hbm, idx_v, hist_v, id_v, hist_sh):
    cid = lax.axis_index("c")
    sid = lax.axis_index("s")
    wid = sid * NC + cid
    pltpu.sync_copy(
        ei_hbm.at[1, pl.ds(wid * EPW, EPW)], idx_v.at[pl.ds(0, EPW)]
    )

    @pl.when(wid < NTAIL)
    def _():
        pltpu.sync_copy(
            ei_hbm.at[1, pl.ds(TAIL + wid * CHUNK, CHUNK)],
            idx_v.at[pl.ds(EPW, CHUNK)],
        )

    zeros16 = jnp.zeros((16,), jnp.float32)
    ones16 = jnp.ones((16,), jnp.float32)

    @pl.loop(0, N_HR)
    def _(i):
        for j in range(8):
            hist_v[i, pl.ds(j * 16, 16)] = zeros16

    @pl.loop(0, N_HR // 16)
    def _(k):
        id_v[pl.ds(k * 16, 16)] = lax.iota(jnp.int32, 16) + k * 16

    @pl.when(sid == 0)
    def _():
        pltpu.sync_copy(hist_v, hist_sh)

    plsc.subcore_barrier()

    def add_hist(e):
        v = idx_v[pl.ds(e * 16, 16)]
        plsc.addupdate_scatter(
            hist_v,
            [lax.shift_right_logical(v, 7), jnp.bitwise_and(v, 127)],
            ones16,
        )

    @pl.loop(0, EPW // 16)
    def _(e):
        add_hist(e)

    @pl.when(wid < NTAIL)
    def _():
        @pl.loop(EPW // 16, (EPW + CHUNK) // 16)
        def _(e):
            add_hist(e)

    pltpu.sync_copy(hist_v, hist_sh.at[id_v], add=True)
    plsc.subcore_barrier()
    pltpu.sync_copy(
        hist_sh.at[pl.ds(sid * 8, 8)], out_hbm.at[cid, pl.ds(sid * 8, 8)]
    )


@functools.partial(
    pl.kernel,
    out_type=jax.ShapeDtypeStruct((NC, N_ACC, D), jnp.float32),
    mesh=_mesh,
    scratch_types=[
        pltpu.VMEM((EPW + CHUNK,), jnp.int32),
        pltpu.VMEM((CHUNK,), jnp.int32),
        pltpu.VMEM((CHUNK,), jnp.int32),
        pltpu.VMEM((CHUNK, D), jnp.float32),
        pltpu.VMEM((CHUNK, D), jnp.float32),
        pltpu.VMEM_SHARED((N_ACC, D), jnp.float32),
        pltpu.SemaphoreType.DMA,
        pltpu.SemaphoreType.DMA,
    ],
)
def _agg_kernel(ei_hbm, g_hbm, zeros_hbm, out_hbm,
                src_v, dst_a, dst_b, rows_a, rows_b, acc_sh, sem_a, sem_b):
    cid = lax.axis_index("c")
    sid = lax.axis_index("s")
    wid = sid * NC + cid
    base = wid * EPW
    pltpu.sync_copy(
        ei_hbm.at[0, pl.ds(base, EPW)], src_v.at[pl.ds(0, EPW)]
    )

    @pl.when(wid < NTAIL)
    def _():
        pltpu.sync_copy(
            ei_hbm.at[0, pl.ds(TAIL + wid * CHUNK, CHUNK)],
            src_v.at[pl.ds(EPW, CHUNK)],
        )

    pltpu.sync_copy(zeros_hbm, acc_sh.at[pl.ds(sid * ZR, ZR)])
    plsc.subcore_barrier()

    def gather(i, rows, sem):
        return pltpu.make_async_copy(
            g_hbm.at[src_v.at[pl.ds(i * CHUNK, CHUNK)]], rows, sem
        )

    def load_dst(i, dst_buf):
        pltpu.sync_copy(ei_hbm.at[1, pl.ds(base + i * CHUNK, CHUNK)], dst_buf)

    load_dst(0, dst_a)
    gather(0, rows_a, sem_a).start()

    @pl.loop(0, CPW // 2)
    def _(k):
        i = k * 2
        load_dst(i + 1, dst_b)
        gather(i + 1, rows_b, sem_b).start()
        gather(i, rows_a, sem_a).wait()
        pltpu.sync_copy(rows_a, acc_sh.at[dst_a], add=True)

        @pl.when(k < CPW // 2 - 1)
        def _():
            load_dst(i + 2, dst_a)
            gather(i + 2, rows_a, sem_a).start()

        gather(i + 1, rows_b, sem_b).wait()
        pltpu.sync_copy(rows_b, acc_sh.at[dst_b], add=True)

    @pl.when(wid < NTAIL)
    def _():
        pltpu.sync_copy(ei_hbm.at[1, pl.ds(TAIL + wid * CHUNK, CHUNK)], dst_a)
        gather(CPW, rows_a, sem_a).start()
        gather(CPW, rows_a, sem_a).wait()
        pltpu.sync_copy(rows_a, acc_sh.at[dst_a], add=True)

    plsc.subcore_barrier()
    pltpu.sync_copy(
        acc_sh.at[pl.ds(sid * ZR, ZR)], out_hbm.at[cid, pl.ds(sid * ZR, ZR)]
    )


def _scale_body(x_ref, w_ref, d0_ref, d1_ref, g_ref, dinv_ref):
    h = jnp.dot(x_ref[...], w_ref[...], preferred_element_type=jnp.float32)
    dinv = lax.rsqrt(d0_ref[...] + d1_ref[...] + 1.0)
    dinv_ref[...] = dinv
    g_ref[...] = h * dinv


def _epilogue_body(dinv_ref, g_ref, acc_ref, b_ref, o_ref):
    o_ref[...] = (
        dinv_ref[...] * (acc_ref[0] + acc_ref[1] + g_ref[...]) + b_ref[...]
    )


_BR = 2000


def kernel(x, edge_index, W, b):
    ei = edge_index.astype(jnp.int32)

    zerosD = jnp.zeros((ZR, D), jnp.float32)

    deg = _deg_kernel(ei).reshape(NC, N_HR * 128)
    d0 = deg[0, :N, None]
    d1 = deg[1, :N, None]

    grid = (N // _BR,)
    row_spec = pl.BlockSpec((_BR, D), lambda i: (i, 0))
    col_spec = pl.BlockSpec((_BR, 1), lambda i: (i, 0))
    g, dinv = pl.pallas_call(
        _scale_body,
        grid=grid,
        in_specs=[
            row_spec,
            pl.BlockSpec((D, D), lambda i: (0, 0)),
            col_spec,
            col_spec,
        ],
        out_specs=[row_spec, col_spec],
        out_shape=[
            jax.ShapeDtypeStruct((N, D), jnp.float32),
            jax.ShapeDtypeStruct((N, 1), jnp.float32),
        ],
    )(x, W, d0, d1)

    acc = _agg_kernel(ei, g, zerosD)

    out = pl.pallas_call(
        _epilogue_body,
        grid=grid,
        in_specs=[
            col_spec,
            row_spec,
            pl.BlockSpec((NC, _BR, D), lambda i: (0, i, 0)),
            pl.BlockSpec((1, D), lambda i: (0, 0)),
        ],
        out_specs=row_spec,
        out_shape=jax.ShapeDtypeStruct((N, D), jnp.float32),
    )(dinv, g, acc, b.reshape(1, D))
    return out


# speedup vs baseline: 1.1640x; 1.0789x over previous
import functools

import jax
import jax.numpy as jnp
from jax import lax
from jax.experimental import pallas as pl
from jax.experimental.pallas import tpu as pltpu
from jax.experimental.pallas import tpu_sc as plsc

N = 10000
D = 128
E = 320000

NC = 2
NS = 16
NW = NC * NS

CHUNK = 128
CPW = 78
EPW = CPW * CHUNK
TAIL = NW * EPW
NTAIL = (E - TAIL) // CHUNK
ZR = 640
N_ACC = NS * ZR

_mesh = plsc.VectorSubcoreMesh(core_axis_name="c", subcore_axis_name="s")


N_HR = 128


@functools.partial(
    pl.kernel,
    out_type=jax.ShapeDtypeStruct((NC, N_HR, 128), jnp.float32),
    mesh=_mesh,
    scratch_types=[
        pltpu.VMEM((EPW + CHUNK,), jnp.int32),
        pltpu.VMEM((N_HR, 128), jnp.float32),
        pltpu.VMEM((N_HR,), jnp.int32),
        pltpu.VMEM_SHARED((N_HR, 128), jnp.float32),
    ],
    compiler_params=pltpu.CompilerParams(needs_layout_passes=False),
)
def _deg_kernel(ei_hbm, out_hbm, idx_v, hist_v, id_v, hist_sh):
    cid = lax.axis_index("c")
    sid = lax.axis_index("s")
    wid = sid * NC + cid
    pltpu.sync_copy(
        ei_hbm.at[1, pl.ds(wid * EPW, EPW)], idx_v.at[pl.ds(0, EPW)]
    )

    @pl.when(wid < NTAIL)
    def _():
        pltpu.sync_copy(
            ei_hbm.at[1, pl.ds(TAIL + wid * CHUNK, CHUNK)],
            idx_v.at[pl.ds(EPW, CHUNK)],
        )

    zeros16 = jnp.zeros((16,), jnp.float32)
    ones16 = jnp.ones((16,), jnp.float32)

    @pl.loop(0, N_HR)
    def _(i):
        for j in range(8):
            hist_v[i, pl.ds(j * 16, 16)] = zeros16

    @pl.loop(0, N_HR // 16)
    def _(k):
        id_v[pl.ds(k * 16, 16)] = lax.iota(jnp.int32, 16) + k * 16

    @pl.when(sid == 0)
    def _():
        pltpu.sync_copy(hist_v, hist_sh)

    plsc.subcore_barrier()

    def add_hist(e):
        v = idx_v[pl.ds(e * 16, 16)]
        plsc.addupdate_scatter(
            hist_v,
            [lax.shift_right_logical(v, 7), jnp.bitwise_and(v, 127)],
            ones16,
        )

    @pl.loop(0, EPW // 16)
    def _(e):
        add_hist(e)

    @pl.when(wid < NTAIL)
    def _():
        @pl.loop(EPW // 16, (EPW + CHUNK) // 16)
        def _(e):
            add_hist(e)

    pltpu.sync_copy(hist_v, hist_sh.at[id_v], add=True)
    plsc.subcore_barrier()
    pltpu.sync_copy(
        hist_sh.at[pl.ds(sid * 8, 8)], out_hbm.at[cid, pl.ds(sid * 8, 8)]
    )


@functools.partial(
    pl.kernel,
    out_type=jax.ShapeDtypeStruct((NC, N_ACC, D), jnp.float32),
    mesh=_mesh,
    scratch_types=[
        pltpu.VMEM((EPW + CHUNK,), jnp.int32),
        pltpu.VMEM((CHUNK,), jnp.int32),
        pltpu.VMEM((CHUNK,), jnp.int32),
        pltpu.VMEM((CHUNK, D), jnp.float32),
        pltpu.VMEM((CHUNK, D), jnp.float32),
        pltpu.VMEM_SHARED((N_ACC, D), jnp.float32),
        pltpu.SemaphoreType.DMA,
        pltpu.SemaphoreType.DMA,
    ],
)
def _agg_kernel(ei_hbm, g_hbm, zeros_hbm, out_hbm,
                src_v, dst_a, dst_b, rows_a, rows_b, acc_sh, sem_a, sem_b):
    cid = lax.axis_index("c")
    sid = lax.axis_index("s")
    wid = sid * NC + cid
    base = wid * EPW
    pltpu.sync_copy(
        ei_hbm.at[0, pl.ds(base, EPW)], src_v.at[pl.ds(0, EPW)]
    )

    @pl.when(wid < NTAIL)
    def _():
        pltpu.sync_copy(
            ei_hbm.at[0, pl.ds(TAIL + wid * CHUNK, CHUNK)],
            src_v.at[pl.ds(EPW, CHUNK)],
        )

    pltpu.sync_copy(zeros_hbm, acc_sh.at[pl.ds(sid * ZR, ZR)])
    plsc.subcore_barrier()

    def gather(i, rows, sem):
        return pltpu.make_async_copy(
            g_hbm.at[src_v.at[pl.ds(i * CHUNK, CHUNK)]], rows, sem
        )

    def load_dst(i, dst_buf):
        pltpu.sync_copy(ei_hbm.at[1, pl.ds(base + i * CHUNK, CHUNK)], dst_buf)

    load_dst(0, dst_a)
    gather(0, rows_a, sem_a).start()

    @pl.loop(0, CPW // 2)
    def _(k):
        i = k * 2
        load_dst(i + 1, dst_b)
        gather(i + 1, rows_b, sem_b).start()
        gather(i, rows_a, sem_a).wait()
        pltpu.sync_copy(rows_a, acc_sh.at[dst_a], add=True)

        @pl.when(k < CPW // 2 - 1)
        def _():
            load_dst(i + 2, dst_a)
            gather(i + 2, rows_a, sem_a).start()

        gather(i + 1, rows_b, sem_b).wait()
        pltpu.sync_copy(rows_b, acc_sh.at[dst_b], add=True)

    @pl.when(wid < NTAIL)
    def _():
        pltpu.sync_copy(ei_hbm.at[1, pl.ds(TAIL + wid * CHUNK, CHUNK)], dst_a)
        gather(CPW, rows_a, sem_a).start()
        gather(CPW, rows_a, sem_a).wait()
        pltpu.sync_copy(rows_a, acc_sh.at[dst_a], add=True)

    plsc.subcore_barrier()
    pltpu.sync_copy(
        acc_sh.at[pl.ds(sid * ZR, ZR)], out_hbm.at[cid, pl.ds(sid * ZR, ZR)]
    )


_BR = 2048
_GRID = (N + _BR - 1) // _BR


def _scale_body(x_ref, w_ref, d_ref, g_ref, dinv_ref):
    h = jnp.dot(x_ref[...], w_ref[...], preferred_element_type=jnp.float32)
    dinv = lax.rsqrt(d_ref[0] + d_ref[1] + 1.0)
    dinv_ref[...] = dinv
    g3 = h.reshape(_BR // 128, 128, D) * dinv[:, :, None]
    g_ref[...] = g3.reshape(_BR, D)


def _epilogue_body(dinv_ref, g_ref, acc_ref, b_ref, o_ref):
    s = acc_ref[0] + acc_ref[1] + g_ref[...]
    o3 = s.reshape(_BR // 128, 128, D) * dinv_ref[...][:, :, None]
    o_ref[...] = o3.reshape(_BR, D) + b_ref[...]


def kernel(x, edge_index, W, b):
    ei = edge_index.astype(jnp.int32)

    zerosD = jnp.zeros((ZR, D), jnp.float32)

    deg3 = _deg_kernel(ei)

    row_spec = pl.BlockSpec((_BR, D), lambda i: (i, 0))
    deg_spec = pl.BlockSpec((NC, _BR // 128, 128), lambda i: (0, i, 0))
    dinv_spec = pl.BlockSpec((_BR // 128, 128), lambda i: (i, 0))
    g, dinv = pl.pallas_call(
        _scale_body,
        grid=(_GRID,),
        in_specs=[
            row_spec,
            pl.BlockSpec((D, D), lambda i: (0, 0)),
            deg_spec,
        ],
        out_specs=[row_spec, dinv_spec],
        out_shape=[
            jax.ShapeDtypeStruct((N, D), jnp.float32),
            jax.ShapeDtypeStruct((_GRID * _BR // 128, 128), jnp.float32),
        ],
    )(x, W, deg3)

    acc = _agg_kernel(ei, g, zerosD)

    out = pl.pallas_call(
        _epilogue_body,
        grid=(_GRID,),
        in_specs=[
            dinv_spec,
            row_spec,
            pl.BlockSpec((NC, _BR, D), lambda i: (0, i, 0)),
            pl.BlockSpec((1, D), lambda i: (0, 0)),
        ],
        out_specs=row_spec,
        out_shape=jax.ShapeDtypeStruct((N, D), jnp.float32),
    )(dinv, g, acc, b.reshape(1, D))
    return out


# speedup vs baseline: 1.2760x; 1.0962x over previous
import functools

import jax
import jax.numpy as jnp
from jax import lax
from jax.experimental import pallas as pl
from jax.experimental.pallas import tpu as pltpu
from jax.experimental.pallas import tpu_sc as plsc

N = 10000
D = 128
E = 320000

NC = 2
NS = 16
NW = NC * NS

CHUNK = 128
CPW = 78
EPW = CPW * CHUNK
TAIL = NW * EPW
NTAIL = (E - TAIL) // CHUNK
ZR = 640
N_ACC = NS * ZR

_mesh = plsc.VectorSubcoreMesh(core_axis_name="c", subcore_axis_name="s")


N_HR = 128


@functools.partial(
    pl.kernel,
    out_type=jax.ShapeDtypeStruct((NC, N_HR, 128), jnp.float32),
    mesh=_mesh,
    scratch_types=[
        pltpu.VMEM((EPW + CHUNK,), jnp.int32),
        pltpu.VMEM((N_HR, 128), jnp.float32),
        pltpu.VMEM((N_HR,), jnp.int32),
        pltpu.VMEM_SHARED((N_HR, 128), jnp.float32),
    ],
    compiler_params=pltpu.CompilerParams(needs_layout_passes=False),
)
def _deg_kernel(ei_hbm, out_hbm, idx_v, hist_v, id_v, hist_sh):
    cid = lax.axis_index("c")
    sid = lax.axis_index("s")
    wid = sid * NC + cid
    pltpu.sync_copy(
        ei_hbm.at[1, pl.ds(wid * EPW, EPW)], idx_v.at[pl.ds(0, EPW)]
    )

    @pl.when(wid < NTAIL)
    def _():
        pltpu.sync_copy(
            ei_hbm.at[1, pl.ds(TAIL + wid * CHUNK, CHUNK)],
            idx_v.at[pl.ds(EPW, CHUNK)],
        )

    zeros16 = jnp.zeros((16,), jnp.float32)
    ones16 = jnp.ones((16,), jnp.float32)

    @pl.loop(0, N_HR)
    def _(i):
        for j in range(8):
            hist_v[i, pl.ds(j * 16, 16)] = zeros16

    @pl.loop(0, N_HR // 16)
    def _(k):
        id_v[pl.ds(k * 16, 16)] = lax.iota(jnp.int32, 16) + k * 16

    @pl.when(sid == 0)
    def _():
        pltpu.sync_copy(hist_v, hist_sh)

    plsc.subcore_barrier()

    def add_hist(e):
        v = idx_v[pl.ds(e * 16, 16)]
        plsc.addupdate_scatter(
            hist_v,
            [lax.shift_right_logical(v, 7), jnp.bitwise_and(v, 127)],
            ones16,
        )

    @pl.loop(0, EPW // 16)
    def _(e):
        add_hist(e)

    @pl.when(wid < NTAIL)
    def _():
        @pl.loop(EPW // 16, (EPW + CHUNK) // 16)
        def _(e):
            add_hist(e)

    pltpu.sync_copy(hist_v, hist_sh.at[id_v], add=True)
    plsc.subcore_barrier()
    pltpu.sync_copy(
        hist_sh.at[pl.ds(sid * 8, 8)], out_hbm.at[cid, pl.ds(sid * 8, 8)]
    )


@functools.partial(
    pl.kernel,
    out_type=jax.ShapeDtypeStruct((NC, N_ACC, D), jnp.float32),
    mesh=_mesh,
    scratch_types=[
        pltpu.VMEM((2, CHUNK), jnp.int32),
        pltpu.VMEM((2, CHUNK), jnp.int32),
        pltpu.VMEM((2, CHUNK), jnp.int32),
        pltpu.VMEM((CHUNK, D), jnp.float32),
        pltpu.VMEM((CHUNK, D), jnp.float32),
        pltpu.VMEM_SHARED((N_ACC, D), jnp.float32),
        pltpu.SemaphoreType.DMA,
        pltpu.SemaphoreType.DMA,
        pltpu.SemaphoreType.DMA,
        pltpu.SemaphoreType.DMA,
        pltpu.SemaphoreType.DMA,
    ],
)
def _agg_kernel(ei_hbm, g_hbm, zeros_hbm, out_hbm,
                p0, p1, p2, rows_a, rows_b, acc_sh,
                sg_a, sg_b, sp0, sp1, sp2):
    cid = lax.axis_index("c")
    sid = lax.axis_index("s")
    wid = sid * NC + cid
    base = wid * EPW
    pairs = (p0, p1, p2)
    psems = (sp0, sp1, sp2)
    rows = (rows_a, rows_b)
    gsems = (sg_a, sg_b)

    pltpu.sync_copy(zeros_hbm, acc_sh.at[pl.ds(sid * ZR, ZR)])
    plsc.subcore_barrier()

    def pload(i, j):
        return pltpu.make_async_copy(
            ei_hbm.at[:, pl.ds(i * CHUNK, CHUNK)], pairs[j], psems[j]
        )

    def gather(j, r):
        return pltpu.make_async_copy(
            g_hbm.at[pairs[j].at[0]], rows[r], gsems[r]
        )

    for j in range(3):
        pload(base // CHUNK + j, j).start()
    for j in range(2):
        pload(0, j).wait()
        gather(j, j).start()

    cbase = base // CHUNK
    not_last = lambda k: k < CPW // 6 - 1

    @pl.loop(0, CPW // 6)
    def _(k):
        i = k * 6
        for j in range(6):
            r, pj = j % 2, j % 3

            def refill(i=i, j=j, pj=pj, r=r):
                pload(cbase + i + j + 3, pj).start()

            def next_gather(i=i, j=j, pj=pj, r=r):
                pload(0, (j + 2) % 3).wait()
                gather((j + 2) % 3, r).start()

            gather(pj, r).wait()
            pltpu.sync_copy(rows[r], acc_sh.at[pairs[pj].at[1]], add=True)
            if j < 3:
                refill()
            else:
                pl.when(not_last(k))(refill)
            if j < 4:
                next_gather()
            else:
                pl.when(not_last(k))(next_gather)

    @pl.when(wid < NTAIL)
    def _():
        pltpu.sync_copy(
            ei_hbm.at[:, pl.ds(TAIL + wid * CHUNK, CHUNK)], pairs[0]
        )
        gather(0, 0).start()
        gather(0, 0).wait()
        pltpu.sync_copy(rows[0], acc_sh.at[pairs[0].at[1]], add=True)

    plsc.subcore_barrier()
    pltpu.sync_copy(
        acc_sh.at[pl.ds(sid * ZR, ZR)], out_hbm.at[cid, pl.ds(sid * ZR, ZR)]
    )


_BR = 2048
_GRID = (N + _BR - 1) // _BR


def _scale_body(x_ref, w_ref, d_ref, g_ref, dinv_ref):
    h = jnp.dot(x_ref[...], w_ref[...], preferred_element_type=jnp.float32)
    dinv = lax.rsqrt(d_ref[0] + d_ref[1] + 1.0)
    dinv_ref[...] = dinv
    g3 = h.reshape(_BR // 128, 128, D) * dinv[:, :, None]
    g_ref[...] = g3.reshape(_BR, D)


def _epilogue_body(dinv_ref, g_ref, acc_ref, b_ref, o_ref):
    s = acc_ref[0] + acc_ref[1] + g_ref[...]
    o3 = s.reshape(_BR // 128, 128, D) * dinv_ref[...][:, :, None]
    o_ref[...] = o3.reshape(_BR, D) + b_ref[...]


def kernel(x, edge_index, W, b):
    ei = edge_index.astype(jnp.int32)

    zerosD = jnp.zeros((ZR, D), jnp.float32)

    deg3 = _deg_kernel(ei)

    row_spec = pl.BlockSpec((_BR, D), lambda i: (i, 0))
    deg_spec = pl.BlockSpec((NC, _BR // 128, 128), lambda i: (0, i, 0))
    dinv_spec = pl.BlockSpec((_BR // 128, 128), lambda i: (i, 0))
    g, dinv = pl.pallas_call(
        _scale_body,
        grid=(_GRID,),
        in_specs=[
            row_spec,
            pl.BlockSpec((D, D), lambda i: (0, 0)),
            deg_spec,
        ],
        out_specs=[row_spec, dinv_spec],
        out_shape=[
            jax.ShapeDtypeStruct((N, D), jnp.float32),
            jax.ShapeDtypeStruct((_GRID * _BR // 128, 128), jnp.float32),
        ],
    )(x, W, deg3)

    acc = _agg_kernel(ei, g, zerosD)

    out = pl.pallas_call(
        _epilogue_body,
        grid=(_GRID,),
        in_specs=[
            dinv_spec,
            row_spec,
            pl.BlockSpec((NC, _BR, D), lambda i: (0, i, 0)),
            pl.BlockSpec((1, D), lambda i: (0, 0)),
        ],
        out_specs=row_spec,
        out_shape=jax.ShapeDtypeStruct((N, D), jnp.float32),
    )(dinv, g, acc, b.reshape(1, D))
    return out


# speedup vs baseline: 1.2814x; 1.0042x over previous
import functools

import jax
import jax.numpy as jnp
from jax import lax
from jax.experimental import pallas as pl
from jax.experimental.pallas import tpu as pltpu
from jax.experimental.pallas import tpu_sc as plsc

N = 10000
D = 128
E = 320000

NC = 2
NS = 16
NW = NC * NS

CHUNK = 128
CPW = 78
EPW = CPW * CHUNK
TAIL = NW * EPW
NTAIL = (E - TAIL) // CHUNK
ZR = 640
N_ACC = NS * ZR

_mesh = plsc.VectorSubcoreMesh(core_axis_name="c", subcore_axis_name="s")


N_HR = 128


@functools.partial(
    pl.kernel,
    out_type=jax.ShapeDtypeStruct((NC, N_HR, 128), jnp.float32),
    mesh=_mesh,
    scratch_types=[
        pltpu.VMEM((EPW + CHUNK,), jnp.int32),
        pltpu.VMEM((N_HR, 128), jnp.float32),
        pltpu.VMEM((N_HR,), jnp.int32),
        pltpu.VMEM_SHARED((N_HR, 128), jnp.float32),
    ],
    compiler_params=pltpu.CompilerParams(needs_layout_passes=False),
)
def _deg_kernel(ei_hbm, out_hbm, idx_v, hist_v, id_v, hist_sh):
    cid = lax.axis_index("c")
    sid = lax.axis_index("s")
    wid = sid * NC + cid
    pltpu.sync_copy(
        ei_hbm.at[1, pl.ds(wid * EPW, EPW)], idx_v.at[pl.ds(0, EPW)]
    )

    @pl.when(wid < NTAIL)
    def _():
        pltpu.sync_copy(
            ei_hbm.at[1, pl.ds(TAIL + wid * CHUNK, CHUNK)],
            idx_v.at[pl.ds(EPW, CHUNK)],
        )

    zeros16 = jnp.zeros((16,), jnp.float32)
    ones16 = jnp.ones((16,), jnp.float32)

    @pl.loop(0, N_HR)
    def _(i):
        for j in range(8):
            hist_v[i, pl.ds(j * 16, 16)] = zeros16

    @pl.loop(0, N_HR // 16)
    def _(k):
        id_v[pl.ds(k * 16, 16)] = lax.iota(jnp.int32, 16) + k * 16

    @pl.when(sid == 0)
    def _():
        pltpu.sync_copy(hist_v, hist_sh)

    plsc.subcore_barrier()

    def add_hist(e):
        v = idx_v[pl.ds(e * 16, 16)]
        plsc.addupdate_scatter(
            hist_v,
            [lax.shift_right_logical(v, 7), jnp.bitwise_and(v, 127)],
            ones16,
        )

    @pl.loop(0, EPW // 16 // 4)
    def _(q):
        for u in range(4):
            add_hist(q * 4 + u)

    @pl.when(wid < NTAIL)
    def _():
        @pl.loop(EPW // 16, (EPW + CHUNK) // 16)
        def _(e):
            add_hist(e)

    pltpu.sync_copy(hist_v, hist_sh.at[id_v], add=True)
    plsc.subcore_barrier()
    pltpu.sync_copy(
        hist_sh.at[pl.ds(sid * 8, 8)], out_hbm.at[cid, pl.ds(sid * 8, 8)]
    )


@functools.partial(
    pl.kernel,
    out_type=jax.ShapeDtypeStruct((NC, N_ACC, D), jnp.float32),
    mesh=_mesh,
    scratch_types=[
        pltpu.VMEM((2, CHUNK), jnp.int32),
        pltpu.VMEM((2, CHUNK), jnp.int32),
        pltpu.VMEM((2, CHUNK), jnp.int32),
        pltpu.VMEM((CHUNK, D), jnp.float32),
        pltpu.VMEM((CHUNK, D), jnp.float32),
        pltpu.VMEM_SHARED((N_ACC, D), jnp.float32),
        pltpu.SemaphoreType.DMA,
        pltpu.SemaphoreType.DMA,
        pltpu.SemaphoreType.DMA,
        pltpu.SemaphoreType.DMA,
        pltpu.SemaphoreType.DMA,
    ],
)
def _agg_kernel(ei_hbm, g_hbm, zeros_hbm, out_hbm,
                p0, p1, p2, rows_a, rows_b, acc_sh,
                sg_a, sg_b, sp0, sp1, sp2):
    cid = lax.axis_index("c")
    sid = lax.axis_index("s")
    wid = sid * NC + cid
    base = wid * EPW
    pairs = (p0, p1, p2)
    psems = (sp0, sp1, sp2)
    rows = (rows_a, rows_b)
    gsems = (sg_a, sg_b)

    pltpu.sync_copy(zeros_hbm, acc_sh.at[pl.ds(sid * ZR, ZR)])
    plsc.subcore_barrier()

    def pload(i, j):
        return pltpu.make_async_copy(
            ei_hbm.at[:, pl.ds(i * CHUNK, CHUNK)], pairs[j], psems[j]
        )

    def gather(j, r):
        return pltpu.make_async_copy(
            g_hbm.at[pairs[j].at[0]], rows[r], gsems[r]
        )

    for j in range(3):
        pload(base // CHUNK + j, j).start()
    for j in range(2):
        pload(0, j).wait()
        gather(j, j).start()

    cbase = base // CHUNK
    not_last = lambda k: k < CPW // 6 - 1

    @pl.loop(0, CPW // 6)
    def _(k):
        i = k * 6
        for j in range(6):
            r, pj = j % 2, j % 3

            def refill(i=i, j=j, pj=pj, r=r):
                pload(cbase + i + j + 3, pj).start()

            def next_gather(i=i, j=j, pj=pj, r=r):
                pload(0, (j + 2) % 3).wait()
                gather((j + 2) % 3, r).start()

            gather(pj, r).wait()
            pltpu.sync_copy(rows[r], acc_sh.at[pairs[pj].at[1]], add=True)
            if j < 3:
                refill()
            else:
                pl.when(not_last(k))(refill)
            if j < 4:
                next_gather()
            else:
                pl.when(not_last(k))(next_gather)

    @pl.when(wid < NTAIL)
    def _():
        pltpu.sync_copy(
            ei_hbm.at[:, pl.ds(TAIL + wid * CHUNK, CHUNK)], pairs[0]
        )
        gather(0, 0).start()
        gather(0, 0).wait()
        pltpu.sync_copy(rows[0], acc_sh.at[pairs[0].at[1]], add=True)

    plsc.subcore_barrier()
    pltpu.sync_copy(
        acc_sh.at[pl.ds(sid * ZR, ZR)], out_hbm.at[cid, pl.ds(sid * ZR, ZR)]
    )


_BR = 2048
_GRID = (N + _BR - 1) // _BR


def _scale_body(x_ref, w_ref, d_ref, g_ref, dinv_ref):
    h = jnp.dot(x_ref[...], w_ref[...], preferred_element_type=jnp.float32)
    dinv = lax.rsqrt(d_ref[0] + d_ref[1] + 1.0)
    dinv_ref[...] = dinv
    g3 = h.reshape(_BR // 128, 128, D) * dinv[:, :, None]
    g_ref[...] = g3.reshape(_BR, D)


def _epilogue_body(dinv_ref, g_ref, acc_ref, b_ref, o_ref):
    s = acc_ref[0] + acc_ref[1] + g_ref[...]
    o3 = s.reshape(_BR // 128, 128, D) * dinv_ref[...][:, :, None]
    o_ref[...] = o3.reshape(_BR, D) + b_ref[...]


def kernel(x, edge_index, W, b):
    ei = edge_index.astype(jnp.int32)

    zerosD = jnp.zeros((ZR, D), jnp.float32)

    deg3 = _deg_kernel(ei)

    row_spec = pl.BlockSpec((_BR, D), lambda i: (i, 0))
    deg_spec = pl.BlockSpec((NC, _BR // 128, 128), lambda i: (0, i, 0))
    dinv_spec = pl.BlockSpec((_BR // 128, 128), lambda i: (i, 0))
    g, dinv = pl.pallas_call(
        _scale_body,
        grid=(_GRID,),
        in_specs=[
            row_spec,
            pl.BlockSpec((D, D), lambda i: (0, 0)),
            deg_spec,
        ],
        out_specs=[row_spec, dinv_spec],
        out_shape=[
            jax.ShapeDtypeStruct((N, D), jnp.float32),
            jax.ShapeDtypeStruct((_GRID * _BR // 128, 128), jnp.float32),
        ],
    )(x, W, deg3)

    acc = _agg_kernel(ei, g, zerosD)

    out = pl.pallas_call(
        _epilogue_body,
        grid=(_GRID,),
        in_specs=[
            dinv_spec,
            row_spec,
            pl.BlockSpec((NC, _BR, D), lambda i: (0, i, 0)),
            pl.BlockSpec((1, D), lambda i: (0, 0)),
        ],
        out_specs=row_spec,
        out_shape=jax.ShapeDtypeStruct((N, D), jnp.float32),
    )(dinv, g, acc, b.reshape(1, D))
    return out
